# baseline (speedup 1.0000x reference)
import functools

import jax
import jax.numpy as jnp
from jax import lax
from jax.experimental import pallas as pl
from jax.experimental.pallas import tpu as pltpu
from jax.experimental.pallas import tpu_sc as plsc

NUM_INPUTS = 4
NUM_EMBEDDINGS = 100000
DIM = 64
LANES = 16
NUM_CORES = 2
NUM_SUBCORES = 16
NW = NUM_CORES * NUM_SUBCORES
PB = 4
AB = 16
LAG = 2
SR = 4
SPLITS = ((0, 128), (128, 72))


@functools.lru_cache(maxsize=None)
def _build(batch: int, seq: int):
    rows_pc = PB * seq
    per_w_b = batch // NW
    per_w = per_w_b * seq
    n_pieces = per_w_b // PB
    pc_blk = AB // PB
    n_blocks = per_w_b // AB
    assert batch % (NW * AB) == 0 and sum(n for _, n in SPLITS) == rows_pc
    mesh = plsc.VectorSubcoreMesh(
        core_axis_name="c", subcore_axis_name="s",
        num_cores=NUM_CORES, num_subcores=NUM_SUBCORES)

    @functools.partial(
        pl.kernel,
        out_type=jax.ShapeDtypeStruct((seq * DIM, batch), jnp.float32),
        mesh=mesh,
        scratch_types=[
            pltpu.VMEM((NUM_INPUTS, per_w), jnp.int32),
            pltpu.VMEM((SR, rows_pc, DIM), jnp.float32),
            pltpu.VMEM((seq * DIM, AB), jnp.float32),
            pltpu.SemaphoreType.DMA,
            pltpu.SemaphoreType.DMA((SR,)),
            pltpu.SemaphoreType.DMA((SR,)),
            pltpu.SemaphoreType.DMA,
        ],
        compiler_params=pltpu.CompilerParams(
            use_tc_tiling_on_sc=False, needs_layout_passes=False),
    )
    def mimo(xf_hbm, tab_hbm, out_hbm, idx_v, stg, acc, isem, g0sem, gasem,
             osem):
        wid = lax.axis_index("s") * NUM_CORES + lax.axis_index("c")
        w_row = wid * per_w
        w_b = wid * per_w_b

        pltpu.async_copy(
            xf_hbm.at[:, pl.ds(w_row, per_w)], idx_v, isem).wait()

        def bias_body(j, _):
            sl = pl.ds(j * LANES, LANES)
            for i in range(1, NUM_INPUTS):
                idx_v[i, sl] = idx_v[i, sl] + (i * NUM_EMBEDDINGS)
            return 0
        lax.fori_loop(0, per_w // LANES, bias_body, 0)

        def gathers(i, p, sb, sem):
            return [
                pltpu.make_async_copy(
                    tab_hbm.at[idx_v.at[i, pl.ds(p * rows_pc + o, n)]],
                    stg.at[sb, pl.ds(o, n)], sem.at[sb])
                for o, n in SPLITS
            ]

        def out_copy(blk):
            return pltpu.make_async_copy(
                acc, out_hbm.at[:, pl.ds(w_b + blk * AB, AB)], osem)

        d_idx = [lax.iota(jnp.int32, LANES) + g * LANES
                 for g in range(DIM // LANES)]

        def body(p, _):
            sb = lax.rem(p, SR)

            @pl.when(jnp.logical_and(p >= LAG, p < n_pieces + LAG))
            def _b():
                pb = p - LAG
                sbb = lax.rem(pb, SR)
                for d in gathers(0, pb, sbb, g0sem):
                    d.wait()
                for i in range(1, NUM_INPUTS):
                    for d in gathers(i, pb, sbb, gasem):
                        d.start(add=True)

            @pl.when(jnp.logical_and(p >= 2 * LAG, p < n_pieces + 2 * LAG))
            def _c():
                pc = p - 2 * LAG
                sbc = lax.rem(pc, SR)
                for i in range(1, NUM_INPUTS):
                    for d in gathers(i, pc, sbc, gasem):
                        d.wait()
                blk = pc // pc_blk
                b_off = (pc % pc_blk) * PB

                @pl.when(jnp.logical_and(pc % pc_blk == 0, pc >= pc_blk))
                def _wait_store():
                    out_copy(blk - 1).wait()

                i_bs = [jnp.full((LANES,), b_off + b_l, jnp.int32)
                        for b_l in range(PB)]

                def scatter_s(s, _):
                    rbase = s * DIM
                    i_rs = [d_idx[g] + rbase for g in range(DIM // LANES)]
                    for b_l in range(PB):
                        row = b_l * seq + s
                        for g in range(DIM // LANES):
                            x = stg[sbc, row, pl.ds(g * LANES, LANES)]
                            plsc.store_scatter(
                                acc, [i_rs[g], i_bs[b_l]], x)
                    return 0
                lax.fori_loop(0, seq, scatter_s, 0)

                @pl.when(pc % pc_blk == pc_blk - 1)
                def _store():
                    out_copy(blk).start()

            @pl.when(p < n_pieces)
            def _a():
                for d in gathers(0, p, sb, g0sem):
                    d.start()
            return 0

        lax.fori_loop(0, n_pieces + 2 * LAG, body, 0)

        out_copy(n_blocks - 1).wait()

    return mimo


def kernel(x, tables):
    num, vocab, dim = tables.shape
    seq = x.shape[-1]
    batch = x.shape[0] // num
    xf = x.reshape(num, batch * seq)
    tf = tables.reshape(num * vocab, dim)
    out = _build(batch, seq)(xf, tf)
    return out.reshape(seq, dim, batch).transpose(2, 0, 1)


# speedup vs baseline: 1.0765x; 1.0765x over previous
import functools

import jax
import jax.numpy as jnp
from jax import lax
from jax.experimental import pallas as pl
from jax.experimental.pallas import tpu as pltpu
from jax.experimental.pallas import tpu_sc as plsc

NUM_INPUTS = 4
NUM_EMBEDDINGS = 100000
DIM = 64
LANES = 16
NUM_CORES = 2
NUM_SUBCORES = 16
NW = NUM_CORES * NUM_SUBCORES
PB = 4
AB = 8
LAG = 2
SR = 4
SPLITS = ((0, 128), (128, 72))


@functools.lru_cache(maxsize=None)
def _build(batch: int, seq: int):
    rows_pc = PB * seq
    per_w_b = batch // NW
    per_w = per_w_b * seq
    n_pieces = per_w_b // PB
    pc_blk = AB // PB
    n_blocks = per_w_b // AB
    assert batch % (NW * AB) == 0 and sum(n for _, n in SPLITS) == rows_pc
    mesh = plsc.VectorSubcoreMesh(
        core_axis_name="c", subcore_axis_name="s",
        num_cores=NUM_CORES, num_subcores=NUM_SUBCORES)

    @functools.partial(
        pl.kernel,
        out_type=jax.ShapeDtypeStruct((seq * DIM, batch), jnp.float32),
        mesh=mesh,
        scratch_types=[
            pltpu.VMEM((NUM_INPUTS, per_w), jnp.int32),
            pltpu.VMEM((SR, rows_pc, DIM), jnp.float32),
            pltpu.VMEM((2, seq * DIM, AB), jnp.float32),
            pltpu.SemaphoreType.DMA,
            pltpu.SemaphoreType.DMA((SR,)),
            pltpu.SemaphoreType.DMA((SR,)),
            pltpu.SemaphoreType.DMA((2,)),
        ],
        compiler_params=pltpu.CompilerParams(
            use_tc_tiling_on_sc=False, needs_layout_passes=False),
    )
    def mimo(xf_hbm, tab_hbm, out_hbm, idx_v, stg, acc, isem, g0sem, gasem,
             osem):
        wid = lax.axis_index("s") * NUM_CORES + lax.axis_index("c")
        w_row = wid * per_w
        w_b = wid * per_w_b

        pltpu.async_copy(
            xf_hbm.at[:, pl.ds(w_row, per_w)], idx_v, isem).wait()

        def bias_body(j, _):
            sl = pl.ds(j * LANES, LANES)
            for i in range(1, NUM_INPUTS):
                idx_v[i, sl] = idx_v[i, sl] + (i * NUM_EMBEDDINGS)
            return 0
        lax.fori_loop(0, per_w // LANES, bias_body, 0)

        def gathers(i, p, sb, sem):
            return [
                pltpu.make_async_copy(
                    tab_hbm.at[idx_v.at[i, pl.ds(p * rows_pc + o, n)]],
                    stg.at[sb, pl.ds(o, n)], sem.at[sb])
                for o, n in SPLITS
            ]

        def out_copy(blk):
            return pltpu.make_async_copy(
                acc.at[blk % 2],
                out_hbm.at[:, pl.ds(w_b + blk * AB, AB)],
                osem.at[blk % 2])

        d_idx = [lax.iota(jnp.int32, LANES) + g * LANES
                 for g in range(DIM // LANES)]

        def body(p, _):
            sb = lax.rem(p, SR)

            @pl.when(jnp.logical_and(p >= LAG, p < n_pieces + LAG))
            def _b():
                pb = p - LAG
                sbb = lax.rem(pb, SR)
                for d in gathers(0, pb, sbb, g0sem):
                    d.wait()
                for i in range(1, NUM_INPUTS):
                    for d in gathers(i, pb, sbb, gasem):
                        d.start(add=True)

            @pl.when(jnp.logical_and(p >= 2 * LAG, p < n_pieces + 2 * LAG))
            def _c():
                pc = p - 2 * LAG
                sbc = lax.rem(pc, SR)
                for i in range(1, NUM_INPUTS):
                    for d in gathers(i, pc, sbc, gasem):
                        d.wait()
                blk = pc // pc_blk
                abuf = blk % 2
                b_off = (pc % pc_blk) * PB

                @pl.when(jnp.logical_and(pc % pc_blk == 0,
                                         pc >= 2 * pc_blk))
                def _wait_store():
                    out_copy(blk - 2).wait()

                i_bs = [jnp.full((LANES,), b_off + b_l, jnp.int32)
                        for b_l in range(PB)]

                def scatter_s(s, _):
                    rbase = s * DIM
                    i_rs = [d_idx[g] + rbase for g in range(DIM // LANES)]
                    for b_l in range(PB):
                        row = b_l * seq + s
                        for g in range(DIM // LANES):
                            x = stg[sbc, row, pl.ds(g * LANES, LANES)]
                            plsc.store_scatter(
                                acc.at[abuf], [i_rs[g], i_bs[b_l]], x)
                    return 0
                lax.fori_loop(0, seq, scatter_s, 0)

                @pl.when(pc % pc_blk == pc_blk - 1)
                def _store():
                    out_copy(blk).start()

            @pl.when(p < n_pieces)
            def _a():
                for d in gathers(0, p, sb, g0sem):
                    d.start()
            return 0

        lax.fori_loop(0, n_pieces + 2 * LAG, body, 0)

        for blk in (n_blocks - 2, n_blocks - 1):
            out_copy(blk).wait()

    return mimo


def kernel(x, tables):
    num, vocab, dim = tables.shape
    seq = x.shape[-1]
    batch = x.shape[0] // num
    xf = x.reshape(num, batch * seq)
    tf = tables.reshape(num * vocab, dim)
    out = _build(batch, seq)(xf, tf)
    return out.reshape(seq, dim, batch).transpose(2, 0, 1)
